# baseline (speedup 1.0000x reference)
import jax
import jax.numpy as jnp
from jax import lax
from jax.experimental import pallas as pl
from jax.experimental.pallas import tpu as pltpu

H = 16
D = 64
BD = 1024
M = 1024
IN = 1024
Z = 64
CH = 512
NBUF = 6
NCHUNK = H * (M // CH)


def _chunk_copy(w_any, bufs, sems, cc):
    hh, c = cc // 2, cc % 2
    return pltpu.make_async_copy(
        w_any.at[hh, pl.ds(c * CH, CH), :], bufs.at[cc % NBUF],
        sems.at[cc % NBUF])


def _body(x_ref, bq_ref, wq_any, bp_any, w_any, out_ref,
          wq_v, bp_v, bufs, sems):
    cp_wq = pltpu.make_async_copy(wq_any, wq_v, sems.at[NBUF])
    cp_bp = pltpu.make_async_copy(bp_any, bp_v, sems.at[NBUF + 1])
    cp_wq.start()
    cp_bp.start()
    for cc in range(NBUF):
        _chunk_copy(w_any, bufs, sems, cc).start()

    cp_wq.wait()
    cp_bp.wait()
    x = x_ref[...]
    s_rows = []
    for i in range(H):
        q = lax.dot_general(x, wq_v[i * D:(i + 1) * D, :],
                            (((1,), (1,)), ((), ())),
                            preferred_element_type=jnp.float32)
        q = q + bq_ref[i:i + 1, :]
        qn = q * lax.rsqrt(jnp.sum(q * q))
        s_rows.append(lax.dot_general(qn, bp_v[i], (((1,), (1,)), ((), ())),
                                      preferred_element_type=jnp.float32))
    s = jnp.concatenate(s_rows, axis=0)

    smax = jnp.max(s, axis=1, keepdims=True)
    hi0 = smax + jnp.maximum(jnp.abs(smax), 1.0) * 1e-6
    lo0 = jnp.min(s, axis=1, keepdims=True)

    def bis(_, carry):
        lo, hi = carry
        mid = 0.5 * (lo + hi)
        cnt = jnp.sum((s >= mid).astype(jnp.int32), axis=1, keepdims=True)
        ge = cnt >= Z
        return (jnp.where(ge, mid, lo), jnp.where(ge, hi, mid))

    lo, _ = lax.fori_loop(0, 48, bis, (lo0, hi0))
    masks = (s >= lo).astype(jnp.float32)

    sub8 = lax.broadcasted_iota(jnp.int32, (8, 1), 0)
    for h in range(H):
        mb = jnp.broadcast_to(masks[h:h + 1, :], (8, BD))
        macc = jnp.full((8, 1), -jnp.inf, jnp.float32)
        midx = jnp.zeros((8, 1), jnp.int32)
        for c in range(M // CH):
            cc = 2 * h + c
            buf = bufs.at[cc % NBUF]
            _chunk_copy(w_any, bufs, sems, cc).wait()

            def grp(g, carry, buf=buf, c=c, mb=mb):
                ma, mi = carry
                wt = buf[pl.ds(g * 8, 8), :]
                part = jnp.sum(wt * mb, axis=1, keepdims=True)
                upd = part > ma
                gid = c * (CH // 8) + g
                return (jnp.where(upd, part, ma), jnp.where(upd, gid, mi))

            macc, midx = lax.fori_loop(0, CH // 8, grp, (macc, midx),
                                       unroll=8)

        amx = jnp.max(macc)
        rows = midx * 8 + sub8
        top = jnp.min(jnp.where(macc == amx, rows, M))

        la = jnp.clip(top, 0, CH - 1)
        lb = jnp.clip(top - CH, 0, CH - 1)
        rowa = bufs.at[(2 * h) % NBUF][pl.ds(la, 1), :]
        rowb = bufs.at[(2 * h + 1) % NBUF][pl.ds(lb, 1), :]
        row = jnp.where(top >= CH, rowb, rowa)

        for cc in (2 * h + NBUF, 2 * h + NBUF + 1):
            if cc < NCHUNK:
                _chunk_copy(w_any, bufs, sems, cc).start()

        o = lax.dot_general(row, bp_v[h], (((1,), (0,)), ((), ())),
                            preferred_element_type=jnp.float32)
        out_ref[h:h + 1, :] = o * (8.0 * lax.rsqrt(jnp.sum(o * o)))


@jax.jit
def _fused(x2, b_q2, W_q, bin_proj, weight_matrix):
    out = pl.pallas_call(
        _body,
        in_specs=[
            pl.BlockSpec(memory_space=pltpu.MemorySpace.VMEM),
            pl.BlockSpec(memory_space=pltpu.MemorySpace.VMEM),
            pl.BlockSpec(memory_space=pltpu.MemorySpace.HBM),
            pl.BlockSpec(memory_space=pltpu.MemorySpace.HBM),
            pl.BlockSpec(memory_space=pltpu.MemorySpace.HBM),
        ],
        out_specs=pl.BlockSpec(memory_space=pltpu.MemorySpace.VMEM),
        out_shape=jax.ShapeDtypeStruct((H, D), jnp.float32),
        scratch_shapes=[
            pltpu.VMEM((IN, IN), jnp.float32),
            pltpu.VMEM((H, BD, D), jnp.float32),
            pltpu.VMEM((NBUF, CH, BD), jnp.float32),
            pltpu.SemaphoreType.DMA((NBUF + 2,)),
        ],
    )(x2, b_q2, W_q, bin_proj, weight_matrix)
    return out


def kernel(x, W_q, b_q, bin_proj, weight_matrix):
    out = _fused(x.reshape(1, IN), b_q.reshape(H, D), W_q, bin_proj,
                 weight_matrix)
    return out.reshape(H * D)


# speedup vs baseline: 1.2866x; 1.2866x over previous
import jax
import jax.numpy as jnp
from jax import lax
from jax.experimental import pallas as pl
from jax.experimental.pallas import tpu as pltpu

H = 16
D = 64
BD = 1024
M = 1024
IN = 1024
Z = 64
CH = 512
NBUF = 6
NCHUNK = H * (M // CH)


def _chunk_copy(w_any, bufs, sems, cc):
    hh, c = cc // 2, cc % 2
    return pltpu.make_async_copy(
        w_any.at[hh, pl.ds(c * CH, CH), :], bufs.at[cc % NBUF],
        sems.at[cc % NBUF])


def _body(x_ref, bq_ref, wq_any, bp_any, w_any, out_ref,
          wq_v, bp_v, bufs, attn_v, sems):
    cp_wq = pltpu.make_async_copy(wq_any, wq_v, sems.at[NBUF])
    cp_bp = pltpu.make_async_copy(bp_any, bp_v, sems.at[NBUF + 1])
    cp_wq.start()
    cp_bp.start()
    for cc in range(NBUF):
        _chunk_copy(w_any, bufs, sems, cc).start()

    cp_wq.wait()
    cp_bp.wait()
    x = x_ref[...]
    s_rows = []
    for i in range(H):
        q = lax.dot_general(x, wq_v[i * D:(i + 1) * D, :],
                            (((1,), (1,)), ((), ())),
                            preferred_element_type=jnp.float32)
        q = q + bq_ref[i:i + 1, :]
        qn = q * lax.rsqrt(jnp.sum(q * q))
        s_rows.append(lax.dot_general(qn, bp_v[i], (((1,), (1,)), ((), ())),
                                      preferred_element_type=jnp.float32))
    s = jnp.concatenate(s_rows, axis=0)

    smax = jnp.max(s, axis=1, keepdims=True)
    hi0 = smax + jnp.maximum(jnp.abs(smax), 1.0) * 1e-6
    lo0 = jnp.min(s, axis=1, keepdims=True)

    def bis(_, carry):
        lo, hi = carry
        mid = 0.5 * (lo + hi)
        cnt = jnp.sum((s >= mid).astype(jnp.int32), axis=1, keepdims=True)
        ge = cnt >= Z
        return (jnp.where(ge, mid, lo), jnp.where(ge, hi, mid))

    lo, _ = lax.fori_loop(0, 48, bis, (lo0, hi0))
    masks = (s >= lo).astype(jnp.float32)

    G = 32
    rowidx = (lax.broadcasted_iota(jnp.int32, (G, M // G), 1) * G
              + lax.broadcasted_iota(jnp.int32, (G, M // G), 0))
    for h in range(H):
        mb = jnp.broadcast_to(masks[h:h + 1, :], (G, BD))
        for c in range(M // CH):
            cc = 2 * h + c
            buf = bufs.at[cc % NBUF]
            _chunk_copy(w_any, bufs, sems, cc).wait()
            for g in range(CH // G):
                wt = buf[G * g:G * (g + 1), :]
                part = jnp.sum(wt * mb, axis=1, keepdims=True)
                mg = c * (CH // G) + g
                attn_v[:, mg:mg + 1] = part

        attn = attn_v[...]
        amx = jnp.max(attn)
        top = jnp.min(jnp.where(attn == amx, rowidx, M))

        la = jnp.clip(top, 0, CH - 1)
        lb = jnp.clip(top - CH, 0, CH - 1)
        rowa = bufs.at[(2 * h) % NBUF][pl.ds(la, 1), :]
        rowb = bufs.at[(2 * h + 1) % NBUF][pl.ds(lb, 1), :]
        row = jnp.where(top >= CH, rowb, rowa)

        for cc in (2 * h + NBUF, 2 * h + NBUF + 1):
            if cc < NCHUNK:
                _chunk_copy(w_any, bufs, sems, cc).start()

        o = lax.dot_general(row, bp_v[h], (((1,), (0,)), ((), ())),
                            preferred_element_type=jnp.float32)
        out_ref[h:h + 1, :] = o * (8.0 * lax.rsqrt(jnp.sum(o * o)))


@jax.jit
def _fused(x2, b_q2, W_q, bin_proj, weight_matrix):
    out = pl.pallas_call(
        _body,
        in_specs=[
            pl.BlockSpec(memory_space=pltpu.MemorySpace.VMEM),
            pl.BlockSpec(memory_space=pltpu.MemorySpace.VMEM),
            pl.BlockSpec(memory_space=pltpu.MemorySpace.HBM),
            pl.BlockSpec(memory_space=pltpu.MemorySpace.HBM),
            pl.BlockSpec(memory_space=pltpu.MemorySpace.HBM),
        ],
        out_specs=pl.BlockSpec(memory_space=pltpu.MemorySpace.VMEM),
        out_shape=jax.ShapeDtypeStruct((H, D), jnp.float32),
        scratch_shapes=[
            pltpu.VMEM((IN, IN), jnp.float32),
            pltpu.VMEM((H, BD, D), jnp.float32),
            pltpu.VMEM((NBUF, CH, BD), jnp.float32),
            pltpu.VMEM((32, M // 32), jnp.float32),
            pltpu.SemaphoreType.DMA((NBUF + 2,)),
        ],
    )(x2, b_q2, W_q, bin_proj, weight_matrix)
    return out


def kernel(x, W_q, b_q, bin_proj, weight_matrix):
    out = _fused(x.reshape(1, IN), b_q.reshape(H, D), W_q, bin_proj,
                 weight_matrix)
    return out.reshape(H * D)


# speedup vs baseline: 1.3195x; 1.0256x over previous
import jax
import jax.numpy as jnp
from jax import lax
from jax.experimental import pallas as pl
from jax.experimental.pallas import tpu as pltpu

H = 16
D = 64
BD = 1024
M = 1024
IN = 1024
Z = 64
NBUF = 4
NCHUNK = H


def _chunk_copy(w_any, bufs, sems, cc):
    return pltpu.make_async_copy(
        w_any.at[cc], bufs.at[cc % NBUF], sems.at[cc % NBUF])


def _body(x_ref, bq_ref, wq_any, bp_any, w_any, out_ref,
          wq_v, bp_v, bufs, attn_v, sems):
    cp_wq = pltpu.make_async_copy(wq_any, wq_v, sems.at[NBUF])
    cp_bp = pltpu.make_async_copy(bp_any, bp_v, sems.at[NBUF + 1])
    cp_wq.start()
    cp_bp.start()
    for cc in range(NBUF):
        _chunk_copy(w_any, bufs, sems, cc).start()

    cp_wq.wait()
    cp_bp.wait()
    x = x_ref[...]
    s_rows = []
    for i in range(H):
        q = lax.dot_general(x, wq_v[i * D:(i + 1) * D, :],
                            (((1,), (1,)), ((), ())),
                            preferred_element_type=jnp.float32)
        q = q + bq_ref[i:i + 1, :]
        qn = q * lax.rsqrt(jnp.sum(q * q))
        s_rows.append(lax.dot_general(qn, bp_v[i], (((1,), (1,)), ((), ())),
                                      preferred_element_type=jnp.float32))
    s = jnp.concatenate(s_rows, axis=0)

    smax = jnp.max(s, axis=1, keepdims=True)
    hi0 = smax + jnp.maximum(jnp.abs(smax), 1.0) * 1e-6
    lo0 = jnp.min(s, axis=1, keepdims=True)

    def bis(_, carry):
        lo, hi = carry
        mid = 0.5 * (lo + hi)
        cnt = jnp.sum((s >= mid).astype(jnp.int32), axis=1, keepdims=True)
        ge = cnt >= Z
        return (jnp.where(ge, mid, lo), jnp.where(ge, hi, mid))

    lo, _ = lax.fori_loop(0, 48, bis, (lo0, hi0))
    masks = (s >= lo).astype(jnp.float32)

    G = 32
    rowidx = (lax.broadcasted_iota(jnp.int32, (G, M // G), 1) * G
              + lax.broadcasted_iota(jnp.int32, (G, M // G), 0))
    for h in range(H):
        mb = jnp.broadcast_to(masks[h:h + 1, :], (G, BD))
        buf = bufs.at[h % NBUF]
        _chunk_copy(w_any, bufs, sems, h).wait()
        for g in range(M // G):
            wt = buf[G * g:G * (g + 1), :]
            part = jnp.sum(wt * mb, axis=1, keepdims=True)
            attn_v[:, g:g + 1] = part

        attn = attn_v[...]
        amx = jnp.max(attn)
        top = jnp.min(jnp.where(attn == amx, rowidx, M))

        row = buf[pl.ds(top, 1), :]

        if h + NBUF < NCHUNK:
            _chunk_copy(w_any, bufs, sems, h + NBUF).start()

        o = lax.dot_general(row, bp_v[h], (((1,), (0,)), ((), ())),
                            preferred_element_type=jnp.float32)
        out_ref[h:h + 1, :] = o * (8.0 * lax.rsqrt(jnp.sum(o * o)))


@jax.jit
def _fused(x2, b_q2, W_q, bin_proj, weight_matrix):
    out = pl.pallas_call(
        _body,
        in_specs=[
            pl.BlockSpec(memory_space=pltpu.MemorySpace.VMEM),
            pl.BlockSpec(memory_space=pltpu.MemorySpace.VMEM),
            pl.BlockSpec(memory_space=pltpu.MemorySpace.HBM),
            pl.BlockSpec(memory_space=pltpu.MemorySpace.HBM),
            pl.BlockSpec(memory_space=pltpu.MemorySpace.HBM),
        ],
        out_specs=pl.BlockSpec(memory_space=pltpu.MemorySpace.VMEM),
        out_shape=jax.ShapeDtypeStruct((H, D), jnp.float32),
        scratch_shapes=[
            pltpu.VMEM((IN, IN), jnp.float32),
            pltpu.VMEM((H, BD, D), jnp.float32),
            pltpu.VMEM((NBUF, M, BD), jnp.float32),
            pltpu.VMEM((32, M // 32), jnp.float32),
            pltpu.SemaphoreType.DMA((NBUF + 2,)),
        ],
    )(x2, b_q2, W_q, bin_proj, weight_matrix)
    return out


def kernel(x, W_q, b_q, bin_proj, weight_matrix):
    out = _fused(x.reshape(1, IN), b_q.reshape(H, D), W_q, bin_proj,
                 weight_matrix)
    return out.reshape(H * D)


# speedup vs baseline: 1.4030x; 1.0633x over previous
import jax
import jax.numpy as jnp
from jax import lax
from jax.experimental import pallas as pl
from jax.experimental.pallas import tpu as pltpu

H = 16
D = 64
BD = 1024
M = 1024
IN = 1024
Z = 64
NBUF = 4


def _slab_copy(w_any, bufs, sems, cc):
    return pltpu.make_async_copy(
        w_any.at[cc], bufs.at[cc % NBUF], sems.at[cc % NBUF])


def _body(x_ref, bq_ref, bp_ref, wq_any, w_any, out_ref,
          wq_v, bufs, attn_v, sems):
    cp_wq = pltpu.make_async_copy(wq_any, wq_v, sems.at[NBUF])
    cp_wq.start()
    for cc in range(NBUF):
        _slab_copy(w_any, bufs, sems, cc).start()

    cp_wq.wait()
    x = x_ref[...].reshape(1, IN)
    s_rows = []
    for i in range(H):
        q = lax.dot_general(x, wq_v[i * D:(i + 1) * D, :],
                            (((1,), (1,)), ((), ())),
                            preferred_element_type=jnp.float32)
        q = q + bq_ref[i * D:(i + 1) * D].reshape(1, D)
        qn = q * lax.rsqrt(jnp.sum(q * q))
        s_rows.append(lax.dot_general(qn, bp_ref[i], (((1,), (1,)), ((), ())),
                                      preferred_element_type=jnp.float32))
    s = jnp.concatenate(s_rows, axis=0)

    smax = jnp.max(s, axis=1, keepdims=True)
    hi0 = smax + jnp.maximum(jnp.abs(smax), 1.0) * 1e-6
    lo0 = jnp.min(s, axis=1, keepdims=True)

    def bis(_, carry):
        lo, hi = carry
        mid = 0.5 * (lo + hi)
        cnt = jnp.sum((s >= mid).astype(jnp.int32), axis=1, keepdims=True)
        ge = cnt >= Z
        return (jnp.where(ge, mid, lo), jnp.where(ge, hi, mid))

    lo, _ = lax.fori_loop(0, 48, bis, (lo0, hi0))
    masks = (s >= lo).astype(jnp.float32)

    G = 32
    rowidx = (lax.broadcasted_iota(jnp.int32, (G, M // G), 1) * G
              + lax.broadcasted_iota(jnp.int32, (G, M // G), 0))
    outs = []
    for h in range(H):
        mb = jnp.broadcast_to(masks[h:h + 1, :], (G, BD))
        buf = bufs.at[h % NBUF]
        _slab_copy(w_any, bufs, sems, h).wait()
        for g in range(M // G):
            wt = buf[G * g:G * (g + 1), :]
            part = jnp.sum(wt * mb, axis=1, keepdims=True)
            attn_v[:, g:g + 1] = part

        attn = attn_v[...]
        amx = jnp.max(attn)
        top = jnp.min(jnp.where(attn == amx, rowidx, M))

        row = buf[pl.ds(top, 1), :]

        if h + NBUF < H:
            _slab_copy(w_any, bufs, sems, h + NBUF).start()

        o = lax.dot_general(row, bp_ref[h], (((1,), (0,)), ((), ())),
                            preferred_element_type=jnp.float32)
        outs.append(o * (8.0 * lax.rsqrt(jnp.sum(o * o))))

    out_ref[...] = jnp.concatenate(outs, axis=1).reshape(H * D)


@jax.jit
def kernel(x, W_q, b_q, bin_proj, weight_matrix):
    return pl.pallas_call(
        _body,
        in_specs=[
            pl.BlockSpec(memory_space=pltpu.MemorySpace.VMEM),
            pl.BlockSpec(memory_space=pltpu.MemorySpace.VMEM),
            pl.BlockSpec(memory_space=pltpu.MemorySpace.VMEM),
            pl.BlockSpec(memory_space=pltpu.MemorySpace.HBM),
            pl.BlockSpec(memory_space=pltpu.MemorySpace.HBM),
        ],
        out_specs=pl.BlockSpec(memory_space=pltpu.MemorySpace.VMEM),
        out_shape=jax.ShapeDtypeStruct((H * D,), jnp.float32),
        scratch_shapes=[
            pltpu.VMEM((IN, IN), jnp.float32),
            pltpu.VMEM((NBUF, M, BD), jnp.float32),
            pltpu.VMEM((32, M // 32), jnp.float32),
            pltpu.SemaphoreType.DMA((NBUF + 1,)),
        ],
    )(x, b_q, bin_proj, W_q, weight_matrix)


# speedup vs baseline: 1.4529x; 1.0355x over previous
import jax
import jax.numpy as jnp
from jax import lax
from jax.experimental import pallas as pl
from jax.experimental.pallas import tpu as pltpu

H = 16
D = 64
BD = 1024
M = 1024
IN = 1024
Z = 64
NBUF = 4


def _slab_copy(w_any, bufs, sems, cc):
    return pltpu.make_async_copy(
        w_any.at[cc], bufs.at[cc % NBUF], sems.at[cc % NBUF])


def _body(x_ref, bq_ref, bp_ref, wq_ref, w_any, out_ref,
          bufs, attn_v, sems):
    for cc in range(NBUF):
        _slab_copy(w_any, bufs, sems, cc).start()

    x = x_ref[...].reshape(1, IN)
    s_rows = []
    for i in range(H):
        q = lax.dot_general(x, wq_ref[i * D:(i + 1) * D, :],
                            (((1,), (1,)), ((), ())),
                            preferred_element_type=jnp.float32)
        q = q + bq_ref[i * D:(i + 1) * D].reshape(1, D)
        qn = q * lax.rsqrt(jnp.sum(q * q))
        s_rows.append(lax.dot_general(qn, bp_ref[i], (((1,), (1,)), ((), ())),
                                      preferred_element_type=jnp.float32))
    s = jnp.concatenate(s_rows, axis=0)

    smax = jnp.max(s, axis=1, keepdims=True)
    hi0 = smax + jnp.maximum(jnp.abs(smax), 1.0) * 1e-6
    lo0 = jnp.min(s, axis=1, keepdims=True)

    def bis(_, carry):
        lo, hi = carry
        mid = 0.5 * (lo + hi)
        cnt = jnp.sum((s >= mid).astype(jnp.int32), axis=1, keepdims=True)
        ge = cnt >= Z
        return (jnp.where(ge, mid, lo), jnp.where(ge, hi, mid))

    lo, _ = lax.fori_loop(0, 48, bis, (lo0, hi0))
    masks = (s >= lo).astype(jnp.float32)

    G = 32
    rowidx = (lax.broadcasted_iota(jnp.int32, (G, M // G), 1) * G
              + lax.broadcasted_iota(jnp.int32, (G, M // G), 0))
    outs = []
    for h in range(H):
        mb = jnp.broadcast_to(masks[h:h + 1, :], (G, BD))
        buf = bufs.at[h % NBUF]
        _slab_copy(w_any, bufs, sems, h).wait()
        for g in range(M // G):
            wt = buf[G * g:G * (g + 1), :]
            part = jnp.sum(wt * mb, axis=1, keepdims=True)
            attn_v[:, g:g + 1] = part

        attn = attn_v[...]
        amx = jnp.max(attn)
        top = jnp.min(jnp.where(attn == amx, rowidx, M))

        row = buf[pl.ds(top, 1), :]

        if h + NBUF < H:
            _slab_copy(w_any, bufs, sems, h + NBUF).start()

        o = lax.dot_general(row, bp_ref[h], (((1,), (0,)), ((), ())),
                            preferred_element_type=jnp.float32)
        outs.append(o * (8.0 * lax.rsqrt(jnp.sum(o * o))))

    out_ref[...] = jnp.concatenate(outs, axis=1).reshape(H * D)


@jax.jit
def kernel(x, W_q, b_q, bin_proj, weight_matrix):
    return pl.pallas_call(
        _body,
        in_specs=[
            pl.BlockSpec(memory_space=pltpu.MemorySpace.VMEM),
            pl.BlockSpec(memory_space=pltpu.MemorySpace.VMEM),
            pl.BlockSpec(memory_space=pltpu.MemorySpace.VMEM),
            pl.BlockSpec(memory_space=pltpu.MemorySpace.VMEM),
            pl.BlockSpec(memory_space=pltpu.MemorySpace.HBM),
        ],
        out_specs=pl.BlockSpec(memory_space=pltpu.MemorySpace.VMEM),
        out_shape=jax.ShapeDtypeStruct((H * D,), jnp.float32),
        scratch_shapes=[
            pltpu.VMEM((NBUF, M, BD), jnp.float32),
            pltpu.VMEM((32, M // 32), jnp.float32),
            pltpu.SemaphoreType.DMA((NBUF,)),
        ],
    )(x, b_q, bin_proj, W_q, weight_matrix)


# speedup vs baseline: 1.7834x; 1.2275x over previous
import jax
import jax.numpy as jnp
from jax import lax
from jax.experimental import pallas as pl
from jax.experimental.pallas import tpu as pltpu

H = 16
D = 64
BD = 1024
M = 1024
IN = 1024
Z = 64
NBUF = 4


def _slab_copy(w_any, bufs, sems, cc):
    return pltpu.make_async_copy(
        w_any.at[cc], bufs.at[cc % NBUF], sems.at[cc % NBUF])


def _body(x_ref, bq_ref, bp_ref, wq_ref, w_any, out_ref,
          bufs, attn_v, sems):
    for cc in range(NBUF):
        _slab_copy(w_any, bufs, sems, cc).start()

    x = x_ref[...].reshape(1, IN)
    s_rows = []
    for i in range(H):
        q = lax.dot_general(x, wq_ref[i * D:(i + 1) * D, :],
                            (((1,), (1,)), ((), ())),
                            preferred_element_type=jnp.float32)
        q = q + bq_ref[i * D:(i + 1) * D].reshape(1, D)
        qn = q * lax.rsqrt(jnp.sum(q * q))
        s_rows.append(lax.dot_general(qn, bp_ref[i], (((1,), (0,)), ((), ())),
                                      preferred_element_type=jnp.float32))
    s = jnp.concatenate(s_rows, axis=0)

    smax = jnp.max(s, axis=1, keepdims=True)
    hi0 = smax + jnp.maximum(jnp.abs(smax), 1.0) * 1e-6
    lo0 = jnp.min(s, axis=1, keepdims=True)

    def bis(_, carry):
        lo, hi = carry
        mid = 0.5 * (lo + hi)
        cnt = jnp.sum((s >= mid).astype(jnp.int32), axis=1, keepdims=True)
        ge = cnt >= Z
        return (jnp.where(ge, mid, lo), jnp.where(ge, hi, mid))

    lo, _ = lax.fori_loop(0, 48, bis, (lo0, hi0))
    masks = (s >= lo).astype(jnp.float32)

    G = 32
    rowidx = (lax.broadcasted_iota(jnp.int32, (G, M // G), 1) * G
              + lax.broadcasted_iota(jnp.int32, (G, M // G), 0))
    outs = []
    for h in range(H):
        mb = jnp.broadcast_to(masks[h:h + 1, :], (G, BD))
        buf = bufs.at[h % NBUF]
        _slab_copy(w_any, bufs, sems, h).wait()
        for g in range(M // G):
            wt = buf[G * g:G * (g + 1), :]
            part = jnp.sum(wt * mb, axis=1, keepdims=True)
            attn_v[:, g:g + 1] = part

        attn = attn_v[...]
        amx = jnp.max(attn)
        top = jnp.min(jnp.where(attn == amx, rowidx, M))

        row = buf[pl.ds(top, 1), :]

        if h + NBUF < H:
            _slab_copy(w_any, bufs, sems, h + NBUF).start()

        o = lax.dot_general(row, bp_ref[h], (((1,), (1,)), ((), ())),
                            preferred_element_type=jnp.float32)
        outs.append(o * (8.0 * lax.rsqrt(jnp.sum(o * o))))

    out_ref[...] = jnp.concatenate(outs, axis=1).reshape(H * D)


@jax.jit
def kernel(x, W_q, b_q, bin_proj, weight_matrix):
    bp_t = jnp.transpose(bin_proj, (0, 2, 1))
    return pl.pallas_call(
        _body,
        in_specs=[
            pl.BlockSpec(memory_space=pltpu.MemorySpace.VMEM),
            pl.BlockSpec(memory_space=pltpu.MemorySpace.VMEM),
            pl.BlockSpec(memory_space=pltpu.MemorySpace.VMEM),
            pl.BlockSpec(memory_space=pltpu.MemorySpace.VMEM),
            pl.BlockSpec(memory_space=pltpu.MemorySpace.HBM),
        ],
        out_specs=pl.BlockSpec(memory_space=pltpu.MemorySpace.VMEM),
        out_shape=jax.ShapeDtypeStruct((H * D,), jnp.float32),
        scratch_shapes=[
            pltpu.VMEM((NBUF, M, BD), jnp.float32),
            pltpu.VMEM((32, M // 32), jnp.float32),
            pltpu.SemaphoreType.DMA((NBUF,)),
        ],
    )(x, b_q, bp_t, W_q, weight_matrix)


# speedup vs baseline: 1.8690x; 1.0480x over previous
import jax
import jax.numpy as jnp
from jax import lax
from jax.experimental import pallas as pl
from jax.experimental.pallas import tpu as pltpu

H = 16
D = 64
BD = 1024
M = 1024
IN = 1024
Z = 64
NBUF = 6


def _slab_copy(w_any, bufs, sems, cc):
    return pltpu.make_async_copy(
        w_any.at[cc], bufs.at[cc % NBUF], sems.at[cc % NBUF])


def _body(x_ref, bq_ref, bp_ref, wq_ref, w_any, out_ref,
          bufs, attn_v, sems):
    for cc in range(NBUF):
        _slab_copy(w_any, bufs, sems, cc).start()

    x = x_ref[...].reshape(1, IN)
    s_rows = []
    for i in range(H):
        q = lax.dot_general(x, wq_ref[i * D:(i + 1) * D, :],
                            (((1,), (1,)), ((), ())),
                            preferred_element_type=jnp.float32)
        q = q + bq_ref[i * D:(i + 1) * D].reshape(1, D)
        qn = q * lax.rsqrt(jnp.sum(q * q))
        s_rows.append(lax.dot_general(qn, bp_ref[i], (((1,), (0,)), ((), ())),
                                      preferred_element_type=jnp.float32))
    s = jnp.concatenate(s_rows, axis=0)

    smax = jnp.max(s, axis=1, keepdims=True)
    hi0 = smax + jnp.maximum(jnp.abs(smax), 1.0) * 1e-6
    lo0 = jnp.min(s, axis=1, keepdims=True)

    def bis(_, carry):
        lo, hi = carry
        mid = 0.5 * (lo + hi)
        cnt = jnp.sum((s >= mid).astype(jnp.int32), axis=1, keepdims=True)
        ge = cnt >= Z
        return (jnp.where(ge, mid, lo), jnp.where(ge, hi, mid))

    lo, _ = lax.fori_loop(0, 48, bis, (lo0, hi0))
    masks = (s >= lo).astype(jnp.float32)

    G = 32
    rowidx = (lax.broadcasted_iota(jnp.int32, (G, M // G), 1) * G
              + lax.broadcasted_iota(jnp.int32, (G, M // G), 0))
    outs = []
    for h in range(H):
        mb = jnp.broadcast_to(masks[h:h + 1, :], (G, BD))
        buf = bufs.at[h % NBUF]
        _slab_copy(w_any, bufs, sems, h).wait()
        for g in range(M // G):
            wt = buf[G * g:G * (g + 1), :]
            part = jnp.sum(wt * mb, axis=1, keepdims=True)
            attn_v[:, g:g + 1] = part

        attn = attn_v[...]
        amx = jnp.max(attn)
        top = jnp.min(jnp.where(attn == amx, rowidx, M))

        row = buf[pl.ds(top, 1), :]

        if h + NBUF < H:
            _slab_copy(w_any, bufs, sems, h + NBUF).start()

        o = lax.dot_general(row, bp_ref[h], (((1,), (1,)), ((), ())),
                            preferred_element_type=jnp.float32)
        outs.append(o * (8.0 * lax.rsqrt(jnp.sum(o * o))))

    out_ref[...] = jnp.concatenate(outs, axis=1).reshape(H * D)


@jax.jit
def kernel(x, W_q, b_q, bin_proj, weight_matrix):
    bp_t = jnp.transpose(bin_proj, (0, 2, 1))
    return pl.pallas_call(
        _body,
        in_specs=[
            pl.BlockSpec(memory_space=pltpu.MemorySpace.VMEM),
            pl.BlockSpec(memory_space=pltpu.MemorySpace.VMEM),
            pl.BlockSpec(memory_space=pltpu.MemorySpace.VMEM),
            pl.BlockSpec(memory_space=pltpu.MemorySpace.VMEM),
            pl.BlockSpec(memory_space=pltpu.MemorySpace.HBM),
        ],
        out_specs=pl.BlockSpec(memory_space=pltpu.MemorySpace.VMEM),
        out_shape=jax.ShapeDtypeStruct((H * D,), jnp.float32),
        scratch_shapes=[
            pltpu.VMEM((NBUF, M, BD), jnp.float32),
            pltpu.VMEM((32, M // 32), jnp.float32),
            pltpu.SemaphoreType.DMA((NBUF,)),
        ],
    )(x, b_q, bp_t, W_q, weight_matrix)


# speedup vs baseline: 1.8754x; 1.0035x over previous
import jax
import jax.numpy as jnp
from jax import lax
from jax.experimental import pallas as pl
from jax.experimental.pallas import tpu as pltpu

H = 16
D = 64
BD = 1024
M = 1024
IN = 1024
Z = 64
NBUF = 8


def _slab_copy(w_any, bufs, sems, cc):
    return pltpu.make_async_copy(
        w_any.at[cc], bufs.at[cc % NBUF], sems.at[cc % NBUF])


def _body(x_ref, bq_ref, bp_ref, wq_ref, w_any, out_ref,
          bufs, attn_v, sems):
    for cc in range(NBUF):
        _slab_copy(w_any, bufs, sems, cc).start()

    x = x_ref[...].reshape(1, IN)
    s_rows = []
    for i in range(H):
        q = lax.dot_general(x, wq_ref[i * D:(i + 1) * D, :],
                            (((1,), (1,)), ((), ())),
                            preferred_element_type=jnp.float32)
        q = q + bq_ref[i * D:(i + 1) * D].reshape(1, D)
        qn = q * lax.rsqrt(jnp.sum(q * q))
        s_rows.append(lax.dot_general(qn, bp_ref[i], (((1,), (0,)), ((), ())),
                                      preferred_element_type=jnp.float32))
    s = jnp.concatenate(s_rows, axis=0)

    smax = jnp.max(s, axis=1, keepdims=True)
    hi0 = smax + jnp.maximum(jnp.abs(smax), 1.0) * 1e-6
    lo0 = jnp.min(s, axis=1, keepdims=True)

    def bis(_, carry):
        lo, hi = carry
        mid = 0.5 * (lo + hi)
        cnt = jnp.sum((s >= mid).astype(jnp.int32), axis=1, keepdims=True)
        ge = cnt >= Z
        return (jnp.where(ge, mid, lo), jnp.where(ge, hi, mid))

    lo, _ = lax.fori_loop(0, 48, bis, (lo0, hi0))
    masks = (s >= lo).astype(jnp.float32)

    G = 32
    rowidx = (lax.broadcasted_iota(jnp.int32, (G, M // G), 1) * G
              + lax.broadcasted_iota(jnp.int32, (G, M // G), 0))
    outs = []
    for h in range(H):
        mb = jnp.broadcast_to(masks[h:h + 1, :], (G, BD))
        buf = bufs.at[h % NBUF]
        _slab_copy(w_any, bufs, sems, h).wait()
        for g in range(M // G):
            wt = buf[G * g:G * (g + 1), :]
            part = jnp.sum(wt * mb, axis=1, keepdims=True)
            attn_v[:, g:g + 1] = part

        attn = attn_v[...]
        amx = jnp.max(attn)
        top = jnp.min(jnp.where(attn == amx, rowidx, M))

        row = buf[pl.ds(top, 1), :]

        if h + NBUF < H:
            _slab_copy(w_any, bufs, sems, h + NBUF).start()

        o = lax.dot_general(row, bp_ref[h], (((1,), (1,)), ((), ())),
                            preferred_element_type=jnp.float32)
        outs.append(o * (8.0 * lax.rsqrt(jnp.sum(o * o))))

    out_ref[...] = jnp.concatenate(outs, axis=1).reshape(H * D)


@jax.jit
def kernel(x, W_q, b_q, bin_proj, weight_matrix):
    bp_t = jnp.transpose(bin_proj, (0, 2, 1))
    return pl.pallas_call(
        _body,
        in_specs=[
            pl.BlockSpec(memory_space=pltpu.MemorySpace.VMEM),
            pl.BlockSpec(memory_space=pltpu.MemorySpace.VMEM),
            pl.BlockSpec(memory_space=pltpu.MemorySpace.VMEM),
            pl.BlockSpec(memory_space=pltpu.MemorySpace.VMEM),
            pl.BlockSpec(memory_space=pltpu.MemorySpace.HBM),
        ],
        out_specs=pl.BlockSpec(memory_space=pltpu.MemorySpace.VMEM),
        out_shape=jax.ShapeDtypeStruct((H * D,), jnp.float32),
        scratch_shapes=[
            pltpu.VMEM((NBUF, M, BD), jnp.float32),
            pltpu.VMEM((32, M // 32), jnp.float32),
            pltpu.SemaphoreType.DMA((NBUF,)),
        ],
    )(x, b_q, bp_t, W_q, weight_matrix)


# speedup vs baseline: 1.8817x; 1.0033x over previous
import jax
import jax.numpy as jnp
from jax import lax
from jax.experimental import pallas as pl
from jax.experimental.pallas import tpu as pltpu

H = 16
D = 64
BD = 1024
M = 1024
IN = 1024
Z = 64
NBUF = 6


def _slab_copy(w_any, bufs, sems, cc):
    return pltpu.make_async_copy(
        w_any.at[cc], bufs.at[cc % NBUF], sems.at[cc % NBUF])


def _body(x_ref, bq_ref, bp_ref, wq_ref, w_any, out_ref,
          bufs, attn_v, sems):
    for cc in range(NBUF):
        _slab_copy(w_any, bufs, sems, cc).start()

    x = x_ref[...].reshape(1, IN)
    s_rows = []
    for i in range(H):
        q = lax.dot_general(x, wq_ref[i * D:(i + 1) * D, :],
                            (((1,), (1,)), ((), ())),
                            preferred_element_type=jnp.float32)
        q = q + bq_ref[i * D:(i + 1) * D].reshape(1, D)
        qn = q * lax.rsqrt(jnp.sum(q * q))
        s_rows.append(lax.dot_general(qn, bp_ref[i], (((1,), (0,)), ((), ())),
                                      preferred_element_type=jnp.float32))
    s = jnp.concatenate(s_rows, axis=0)

    smax = jnp.max(s, axis=1, keepdims=True)
    hi0 = smax + jnp.maximum(jnp.abs(smax), 1.0) * 1e-6
    lo0 = jnp.min(s, axis=1, keepdims=True)

    def bis(_, carry):
        lo, hi = carry
        mid = 0.5 * (lo + hi)
        cnt = jnp.sum((s >= mid).astype(jnp.int32), axis=1, keepdims=True)
        ge = cnt >= Z
        return (jnp.where(ge, mid, lo), jnp.where(ge, hi, mid))

    lo, _ = lax.fori_loop(0, 48, bis, (lo0, hi0))
    masks = (s >= lo).astype(jnp.float32)

    G = 32
    rowidx = (lax.broadcasted_iota(jnp.int32, (G, M // G), 1) * G
              + lax.broadcasted_iota(jnp.int32, (G, M // G), 0))
    outs = []
    for h in range(H):
        mb = jnp.broadcast_to(masks[h:h + 1, :], (G, BD))
        buf = bufs.at[h % NBUF]
        _slab_copy(w_any, bufs, sems, h).wait()
        for g in range(M // G):
            wt = buf[G * g:G * (g + 1), :]
            part = jnp.sum(wt * mb, axis=1, keepdims=True)
            attn_v[:, g:g + 1] = part

        attn = attn_v[...]
        amx = jnp.max(attn)
        top = jnp.min(jnp.where(attn == amx, rowidx, M))

        row = buf[pl.ds(top, 1), :]

        if h + NBUF < H:
            _slab_copy(w_any, bufs, sems, h + NBUF).start()

        o = lax.dot_general(row, bp_ref[h], (((1,), (1,)), ((), ())),
                            preferred_element_type=jnp.float32)
        outs.append(o * (8.0 * lax.rsqrt(jnp.sum(o * o))))

    out_ref[...] = jnp.concatenate(outs, axis=1).reshape(H * D)


@jax.jit
def kernel(x, W_q, b_q, bin_proj, weight_matrix):
    bp_t = jnp.transpose(bin_proj, (0, 2, 1))
    return pl.pallas_call(
        _body,
        in_specs=[
            pl.BlockSpec(memory_space=pltpu.MemorySpace.VMEM),
            pl.BlockSpec(memory_space=pltpu.MemorySpace.VMEM),
            pl.BlockSpec(memory_space=pltpu.MemorySpace.VMEM),
            pl.BlockSpec(memory_space=pltpu.MemorySpace.VMEM),
            pl.BlockSpec(memory_space=pltpu.MemorySpace.HBM),
        ],
        out_specs=pl.BlockSpec(memory_space=pltpu.MemorySpace.VMEM),
        out_shape=jax.ShapeDtypeStruct((H * D,), jnp.float32),
        scratch_shapes=[
            pltpu.VMEM((NBUF, M, BD), jnp.float32),
            pltpu.VMEM((32, M // 32), jnp.float32),
            pltpu.SemaphoreType.DMA((NBUF,)),
        ],
    )(x, b_q, bp_t, W_q, weight_matrix)
